# Initial kernel scaffold; baseline (speedup 1.0000x reference)
import jax
import jax.numpy as jnp
from jax.experimental import pallas as pl


def kernel(x, d, grid, opacity):
    raise NotImplementedError("write your pallas kernel here")


# speedup vs baseline: 3.6891x; 3.6891x over previous
import functools

import jax
import jax.numpy as jnp
from jax import lax
from jax.experimental import pallas as pl
from jax.experimental.pallas import tpu as pltpu
from jax.experimental.pallas import tpu_sc as plsc

IDIM = 128
NB_SAMPLES = 32
NG = IDIM + 1
NIJ = NG * NG
QROWS = 3
ROWF = 128
KPAD = 63 * (QROWS - 1) + ROWF // 2

NC = 2
NS = 16
NW = NC * NS
LANES = 16
CHUNK = 128
GPER = CHUNK * 4 // 128


_TBL_B1 = 3


def _tbl_body(g_ref, o_ref, out_ref):
    g = g_ref[...]
    h = jnp.sum(g, axis=-1)
    o = o_ref[...]
    m = lax.broadcasted_iota(jnp.int32, (KPAD, QROWS * ROWF), 0)
    n = lax.broadcasted_iota(jnp.int32, (KPAD, QROWS * ROWF), 1)
    e = n % ROWF
    q = n // ROWF
    kk = 63 * q + e // 2
    selh = ((e % 2 == 0) & (m == kk)).astype(jnp.float32)
    selo = ((e % 2 == 1) & (m == kk)).astype(jnp.float32)
    z = jnp.zeros((NG, KPAD - NG), jnp.float32)
    for b in range(_TBL_B1):
        hp = jnp.concatenate([h[b], z], axis=-1)
        op = jnp.concatenate([o[b], z], axis=-1)
        out_ref[b] = (jnp.dot(hp, selh, preferred_element_type=jnp.float32)
                      + jnp.dot(op, selo, preferred_element_type=jnp.float32))


def _build_table(grid, opacity):
    return pl.pallas_call(
        _tbl_body,
        grid=(NG // _TBL_B1,),
        in_specs=[
            pl.BlockSpec((_TBL_B1, NG, NG, 9), lambda b: (b, 0, 0, 0)),
            pl.BlockSpec((_TBL_B1, NG, NG), lambda b: (b, 0, 0)),
        ],
        out_specs=pl.BlockSpec((_TBL_B1, NG, QROWS * ROWF), lambda b: (b, 0, 0)),
        out_shape=jax.ShapeDtypeStruct((NG, NG, QROWS * ROWF), jnp.float32),
    )(grid, opacity)


def _sc_body(tbl, idx2, w00, w01, w10, w11, fzv, ppv,
             outh, outo,
             idx_v, rows_v, w00_v, w01_v, w10_v, w11_v, fz_v, pp_v,
             oh_v, oo_v, sem):
    wid = lax.axis_index("s") * NC + lax.axis_index("c")
    npw = outh.shape[0] // NW
    nch = npw // CHUNK

    def chunk_body(g, _):
        s0 = wid * npw + g * CHUNK
        pltpu.sync_copy(idx2.at[pl.ds(s0 * 4, CHUNK * 4)], idx_v)
        pltpu.sync_copy(w00.at[pl.ds(s0, CHUNK)], w00_v)
        pltpu.sync_copy(w01.at[pl.ds(s0, CHUNK)], w01_v)
        pltpu.sync_copy(w10.at[pl.ds(s0, CHUNK)], w10_v)
        pltpu.sync_copy(w11.at[pl.ds(s0, CHUNK)], w11_v)
        pltpu.sync_copy(fzv.at[pl.ds(s0, CHUNK)], fz_v)
        pltpu.sync_copy(ppv.at[pl.ds(s0, CHUNK)], pp_v)
        copies = [
            pltpu.async_copy(tbl.at[idx_v.at[pl.ds(r * 128, 128)]],
                             rows_v.at[pl.ds(r * 128, 128)], sem)
            for r in range(GPER)
        ]
        for c in copies:
            c.wait()

        wrefs = (w00_v, w01_v, w10_v, w11_v)

        def sub_body(t, _):
            sl = pl.ds(t * LANES, LANES)
            pv = pp_v[sl]
            fz = fz_v[sl]
            lanes = lax.broadcasted_iota(jnp.int32, (LANES,), 0)
            rbase = (t * LANES + lanes) * 4
            zz = lanes * 0
            ev = pv + pv
            acc_h = jnp.zeros((LANES,), jnp.float32)
            acc_o = jnp.zeros((LANES,), jnp.float32)
            for c in range(4):
                rvec = rbase + c
                h0 = plsc.load_gather(rows_v, [rvec, zz, ev])
                h1 = plsc.load_gather(rows_v, [rvec, zz, ev + 2])
                o0 = plsc.load_gather(rows_v, [rvec, zz, ev + 1])
                o1 = plsc.load_gather(rows_v, [rvec, zz, ev + 3])
                w = wrefs[c][sl]
                acc_h = acc_h + w * (h0 + fz * (h1 - h0))
                acc_o = acc_o + w * (o0 + fz * (o1 - o0))
            oh_v[sl] = acc_h
            oo_v[sl] = acc_o
            return _

        lax.fori_loop(0, CHUNK // LANES, sub_body, None)
        pltpu.sync_copy(oh_v, outh.at[pl.ds(s0, CHUNK)])
        pltpu.sync_copy(oo_v, outo.at[pl.ds(s0, CHUNK)])
        return _

    lax.fori_loop(0, nch, chunk_body, None)


def _sc_gather(tbl, idx2, w00, w01, w10, w11, fzv, ppv, n):
    mesh = plsc.VectorSubcoreMesh(core_axis_name="c", subcore_axis_name="s",
                                  num_cores=NC, num_subcores=NS)
    f = pl.kernel(
        _sc_body,
        out_type=(jax.ShapeDtypeStruct((n,), jnp.float32),
                  jax.ShapeDtypeStruct((n,), jnp.float32)),
        mesh=mesh,
        scratch_types=[
            pltpu.VMEM((CHUNK * 4,), jnp.int32),
            pltpu.VMEM((CHUNK * 4, 1, ROWF), jnp.float32),
            pltpu.VMEM((CHUNK,), jnp.float32),
            pltpu.VMEM((CHUNK,), jnp.float32),
            pltpu.VMEM((CHUNK,), jnp.float32),
            pltpu.VMEM((CHUNK,), jnp.float32),
            pltpu.VMEM((CHUNK,), jnp.float32),
            pltpu.VMEM((CHUNK,), jnp.int32),
            pltpu.VMEM((CHUNK,), jnp.float32),
            pltpu.VMEM((CHUNK,), jnp.float32),
            pltpu.SemaphoreType.DMA,
        ],
        compiler_params=pltpu.CompilerParams(needs_layout_passes=False),
    )
    return f(tbl, idx2, w00, w01, w10, w11, fzv, ppv)


def _render_body(h_ref, o_ref, s_ref, out_ref):
    hh = h_ref[...]
    oo = o_ref[...]
    ss = s_ref[...]
    S1 = NB_SAMPLES - 1
    delta = ss[:, 1:] - ss[:, :S1]
    cur = delta * oo[:, :S1]
    t_i = lax.broadcasted_iota(jnp.int32, (S1, S1), 0)
    s_i = lax.broadcasted_iota(jnp.int32, (S1, S1), 1)
    excl = (t_i < s_i).astype(jnp.float32)
    cumm = jnp.dot(cur, excl, preferred_element_type=jnp.float32)
    trans = jnp.exp(-cumm)
    cs = 1.0 / (1.0 + jnp.exp(-hh[:, :S1]))
    res = jnp.sum(trans * (1.0 - jnp.exp(-cur)) * cs, axis=1)
    out_ref[...] = res.reshape(1, 1, -1)


def _render(interp_h, interp_o, samples, nb_rays):
    RB = 2048
    out = pl.pallas_call(
        _render_body,
        grid=(nb_rays // RB,),
        in_specs=[
            pl.BlockSpec((RB, NB_SAMPLES), lambda b: (b, 0)),
            pl.BlockSpec((RB, NB_SAMPLES), lambda b: (b, 0)),
            pl.BlockSpec((RB, NB_SAMPLES), lambda b: (b, 0)),
        ],
        out_specs=pl.BlockSpec((1, 1, RB), lambda b: (b, 0, 0)),
        out_shape=jax.ShapeDtypeStruct((nb_rays // RB, 1, RB), jnp.float32),
    )(interp_h, interp_o, samples)
    return out.reshape(nb_rays)


@jax.jit
def kernel(x, d, grid, opacity):
    nb_rays = x.shape[0]
    n = nb_rays * NB_SAMPLES

    inv_d = 1.0 / d
    inf = float(IDIM) * IDIM * IDIM
    t0 = (0.0 - x) * inv_d
    t1 = (float(IDIM) - x) * inv_d
    tmin = jnp.maximum(-inf, jnp.max(jnp.minimum(t0, t1), axis=1))
    tmax = jnp.minimum(inf, jnp.min(jnp.maximum(t0, t1), axis=1))
    u = jax.random.uniform(jax.random.key(42), (NB_SAMPLES, nb_rays),
                           jnp.float32)
    samples = jnp.sort((tmin[None, :] + u * (tmax - tmin)[None, :]).T, axis=1)

    pts = x[:, None, :] + samples[:, :, None] * d[:, None, :]
    fl = jnp.floor(pts)
    base = jnp.clip(fl.astype(jnp.int32), 0, IDIM - 1)
    frac = pts - fl
    fx, fy, fz = frac[..., 0], frac[..., 1], frac[..., 2]
    bi, bj, bk = base[..., 0], base[..., 1], base[..., 2]
    q = bk // 63
    p = bk - 63 * q

    ij = bi * NG + bj
    rowidx = jnp.stack(
        [(ij + di * NG + dj) * QROWS + q
         for di in (0, 1) for dj in (0, 1)], axis=-1)
    idx2 = rowidx.reshape(n * 4)

    w00 = ((1 - fx) * (1 - fy)).reshape(n)
    w01 = ((1 - fx) * fy).reshape(n)
    w10 = (fx * (1 - fy)).reshape(n)
    w11 = (fx * fy).reshape(n)
    fzf = fz.reshape(n)
    pf = p.reshape(n)

    tbl = _build_table(grid, opacity).reshape(NIJ * QROWS, 1, ROWF)

    interp_h, interp_o = _sc_gather(tbl, idx2, w00, w01, w10, w11, fzf, pf, n)

    return _render(interp_h.reshape(nb_rays, NB_SAMPLES),
                   interp_o.reshape(nb_rays, NB_SAMPLES),
                   samples, nb_rays)


# speedup vs baseline: 4.2908x; 1.1631x over previous
import functools

import jax
import jax.numpy as jnp
from jax import lax
from jax.experimental import pallas as pl
from jax.experimental.pallas import tpu as pltpu
from jax.experimental.pallas import tpu_sc as plsc

IDIM = 128
NB_SAMPLES = 32
NG = IDIM + 1
NIJ = NG * NG
QROWS = 3
ROWF = 128
KPAD = 63 * (QROWS - 1) + ROWF // 2

NC = 2
NS = 16
NW = NC * NS
LANES = 16
CHUNK = 128
GPER = CHUNK * 4 // 128


_TBL_B1 = 3


def _tbl_body(g_ref, o_ref, out_ref):
    b = pl.program_id(0)
    g = g_ref[...]
    o = o_ref[...]
    ms = lax.broadcasted_iota(jnp.int32, (NG * 9, NG), 0)
    ks = lax.broadcasted_iota(jnp.int32, (NG * 9, NG), 1)
    ssum = (ms // 9 == ks).astype(jnp.float32)
    h = jnp.stack([jnp.dot(g[bs], ssum, preferred_element_type=jnp.float32)
                   for bs in range(_TBL_B1)])
    m = lax.broadcasted_iota(jnp.int32, (KPAD, QROWS * ROWF), 0)
    n = lax.broadcasted_iota(jnp.int32, (KPAD, QROWS * ROWF), 1)
    e = n % ROWF
    q = n // ROWF
    kk = 63 * q + e // 2
    selh = ((e % 2 == 0) & (m == kk)).astype(jnp.float32)
    selo = ((e % 2 == 1) & (m == kk)).astype(jnp.float32)
    z = jnp.zeros((NG, KPAD - NG), jnp.float32)
    for bs in range(_TBL_B1):
        hp = jnp.concatenate([h[bs], z], axis=-1)
        op = jnp.concatenate([o[bs], z], axis=-1)
        mm = (jnp.dot(hp, selh, preferred_element_type=jnp.float32)
              + jnp.dot(op, selo, preferred_element_type=jnp.float32))
        for qq in range(QROWS):
            out_ref[pl.ds((qq * NG + b * _TBL_B1 + bs) * NG, NG), :] = (
                mm[:, qq * ROWF:(qq + 1) * ROWF])


def _build_table(grid, opacity):
    return pl.pallas_call(
        _tbl_body,
        grid=(NG // _TBL_B1,),
        in_specs=[
            pl.BlockSpec((_TBL_B1, NG, NG * 9), lambda b: (b, 0, 0)),
            pl.BlockSpec((_TBL_B1, NG, NG), lambda b: (b, 0, 0)),
        ],
        out_specs=pl.BlockSpec((NIJ * QROWS, ROWF), lambda b: (0, 0)),
        out_shape=jax.ShapeDtypeStruct((NIJ * QROWS, ROWF), jnp.float32),
    )(grid.reshape(NG, NG, NG * 9), opacity)


def _sc_body(tbl, idx2, w00, w01, w10, w11, fzv, ppv,
             outh, outo,
             idx_v, rows_v, w00_v, w01_v, w10_v, w11_v, fz_v, pp_v,
             oh_v, oo_v, sem):
    wid = lax.axis_index("s") * NC + lax.axis_index("c")
    npw = outh.shape[0] // NW
    nch = npw // CHUNK

    def chunk_body(g, _):
        s0 = wid * npw + g * CHUNK
        pltpu.sync_copy(idx2.at[pl.ds(s0 * 4, CHUNK * 4)], idx_v)
        pltpu.sync_copy(w00.at[pl.ds(s0, CHUNK)], w00_v)
        pltpu.sync_copy(w01.at[pl.ds(s0, CHUNK)], w01_v)
        pltpu.sync_copy(w10.at[pl.ds(s0, CHUNK)], w10_v)
        pltpu.sync_copy(w11.at[pl.ds(s0, CHUNK)], w11_v)
        pltpu.sync_copy(fzv.at[pl.ds(s0, CHUNK)], fz_v)
        pltpu.sync_copy(ppv.at[pl.ds(s0, CHUNK)], pp_v)
        copies = [
            pltpu.async_copy(tbl.at[idx_v.at[pl.ds(r * 128, 128)]],
                             rows_v.at[pl.ds(r * 128, 128)], sem)
            for r in range(GPER)
        ]
        for c in copies:
            c.wait()

        wrefs = (w00_v, w01_v, w10_v, w11_v)

        def sub_body(t, _):
            sl = pl.ds(t * LANES, LANES)
            pv = pp_v[sl]
            fz = fz_v[sl]
            lanes = lax.broadcasted_iota(jnp.int32, (LANES,), 0)
            rbase = (t * LANES + lanes) * 4
            zz = lanes * 0
            ev = pv + pv
            acc_h = jnp.zeros((LANES,), jnp.float32)
            acc_o = jnp.zeros((LANES,), jnp.float32)
            for c in range(4):
                rvec = rbase + c
                h0 = plsc.load_gather(rows_v, [rvec, zz, ev])
                h1 = plsc.load_gather(rows_v, [rvec, zz, ev + 2])
                o0 = plsc.load_gather(rows_v, [rvec, zz, ev + 1])
                o1 = plsc.load_gather(rows_v, [rvec, zz, ev + 3])
                w = wrefs[c][sl]
                acc_h = acc_h + w * (h0 + fz * (h1 - h0))
                acc_o = acc_o + w * (o0 + fz * (o1 - o0))
            oh_v[sl] = acc_h
            oo_v[sl] = acc_o
            return _

        lax.fori_loop(0, CHUNK // LANES, sub_body, None)
        pltpu.sync_copy(oh_v, outh.at[pl.ds(s0, CHUNK)])
        pltpu.sync_copy(oo_v, outo.at[pl.ds(s0, CHUNK)])
        return _

    lax.fori_loop(0, nch, chunk_body, None)


def _sc_gather(tbl, idx2, w00, w01, w10, w11, fzv, ppv, n):
    mesh = plsc.VectorSubcoreMesh(core_axis_name="c", subcore_axis_name="s",
                                  num_cores=NC, num_subcores=NS)
    f = pl.kernel(
        _sc_body,
        out_type=(jax.ShapeDtypeStruct((n,), jnp.float32),
                  jax.ShapeDtypeStruct((n,), jnp.float32)),
        mesh=mesh,
        scratch_types=[
            pltpu.VMEM((CHUNK * 4,), jnp.int32),
            pltpu.VMEM((CHUNK * 4, 1, ROWF), jnp.float32),
            pltpu.VMEM((CHUNK,), jnp.float32),
            pltpu.VMEM((CHUNK,), jnp.float32),
            pltpu.VMEM((CHUNK,), jnp.float32),
            pltpu.VMEM((CHUNK,), jnp.float32),
            pltpu.VMEM((CHUNK,), jnp.float32),
            pltpu.VMEM((CHUNK,), jnp.int32),
            pltpu.VMEM((CHUNK,), jnp.float32),
            pltpu.VMEM((CHUNK,), jnp.float32),
            pltpu.SemaphoreType.DMA,
        ],
        compiler_params=pltpu.CompilerParams(needs_layout_passes=False),
    )
    return f(tbl, idx2, w00, w01, w10, w11, fzv, ppv)


def _render_body(h_ref, o_ref, s_ref, out_ref):
    hh = h_ref[...]
    oo = o_ref[...]
    ss = s_ref[...]
    S1 = NB_SAMPLES - 1
    delta = ss[:, 1:] - ss[:, :S1]
    cur = delta * oo[:, :S1]
    t_i = lax.broadcasted_iota(jnp.int32, (S1, S1), 0)
    s_i = lax.broadcasted_iota(jnp.int32, (S1, S1), 1)
    excl = (t_i < s_i).astype(jnp.float32)
    cumm = jnp.dot(cur, excl, preferred_element_type=jnp.float32)
    trans = jnp.exp(-cumm)
    cs = 1.0 / (1.0 + jnp.exp(-hh[:, :S1]))
    res = jnp.sum(trans * (1.0 - jnp.exp(-cur)) * cs, axis=1)
    out_ref[...] = res.reshape(1, 1, -1)


def _render(interp_h, interp_o, samples, nb_rays):
    RB = 2048
    out = pl.pallas_call(
        _render_body,
        grid=(nb_rays // RB,),
        in_specs=[
            pl.BlockSpec((RB, NB_SAMPLES), lambda b: (b, 0)),
            pl.BlockSpec((RB, NB_SAMPLES), lambda b: (b, 0)),
            pl.BlockSpec((RB, NB_SAMPLES), lambda b: (b, 0)),
        ],
        out_specs=pl.BlockSpec((1, 1, RB), lambda b: (b, 0, 0)),
        out_shape=jax.ShapeDtypeStruct((nb_rays // RB, 1, RB), jnp.float32),
    )(interp_h, interp_o, samples)
    return out.reshape(nb_rays)


@jax.jit
def kernel(x, d, grid, opacity):
    nb_rays = x.shape[0]
    n = nb_rays * NB_SAMPLES

    inv_d = 1.0 / d
    inf = float(IDIM) * IDIM * IDIM
    t0 = (0.0 - x) * inv_d
    t1 = (float(IDIM) - x) * inv_d
    tmin = jnp.maximum(-inf, jnp.max(jnp.minimum(t0, t1), axis=1))
    tmax = jnp.minimum(inf, jnp.min(jnp.maximum(t0, t1), axis=1))
    u = jax.random.uniform(jax.random.key(42), (NB_SAMPLES, nb_rays),
                           jnp.float32)
    samples = jnp.sort((tmin[None, :] + u * (tmax - tmin)[None, :]).T, axis=1)

    pts = x[:, None, :] + samples[:, :, None] * d[:, None, :]
    fl = jnp.floor(pts)
    base = jnp.clip(fl.astype(jnp.int32), 0, IDIM - 1)
    frac = pts - fl
    fx, fy, fz = frac[..., 0], frac[..., 1], frac[..., 2]
    bi, bj, bk = base[..., 0], base[..., 1], base[..., 2]
    q = bk // 63
    p = bk - 63 * q

    ij = bi * NG + bj
    rowidx = jnp.stack(
        [q * NIJ + ij + di * NG + dj
         for di in (0, 1) for dj in (0, 1)], axis=-1)
    idx2 = rowidx.reshape(n * 4)

    w00 = ((1 - fx) * (1 - fy)).reshape(n)
    w01 = ((1 - fx) * fy).reshape(n)
    w10 = (fx * (1 - fy)).reshape(n)
    w11 = (fx * fy).reshape(n)
    fzf = fz.reshape(n)
    pf = p.reshape(n)

    tbl = _build_table(grid, opacity).reshape(NIJ * QROWS, 1, ROWF)


    interp_h, interp_o = _sc_gather(tbl, idx2, w00, w01, w10, w11, fzf, pf, n)

    return _render(interp_h.reshape(nb_rays, NB_SAMPLES),
                   interp_o.reshape(nb_rays, NB_SAMPLES),
                   samples, nb_rays)


# speedup vs baseline: 7.1603x; 1.6688x over previous
import functools

import jax
import jax.numpy as jnp
from jax import lax
from jax.experimental import pallas as pl
from jax.experimental.pallas import tpu as pltpu
from jax.experimental.pallas import tpu_sc as plsc

IDIM = 128
NB_SAMPLES = 32
NG = IDIM + 1
NIJ = NG * NG
QROWS = 3
ROWF = 128
KPAD = 63 * (QROWS - 1) + ROWF // 2

NC = 2
NS = 16
NW = NC * NS
LANES = 16
CHUNK = 128
GPER = CHUNK * 4 // 128


_TBL_B1 = 3


def _tbl_body(g_ref, o_ref, out_ref):
    b = pl.program_id(0)
    g = g_ref[...]
    o = o_ref[...]
    ms = lax.broadcasted_iota(jnp.int32, (NG * 9, NG), 0)
    ks = lax.broadcasted_iota(jnp.int32, (NG * 9, NG), 1)
    ssum = (ms // 9 == ks).astype(jnp.float32)
    h = jnp.stack([jnp.dot(g[bs], ssum, preferred_element_type=jnp.float32)
                   for bs in range(_TBL_B1)])
    m = lax.broadcasted_iota(jnp.int32, (KPAD, QROWS * ROWF), 0)
    n = lax.broadcasted_iota(jnp.int32, (KPAD, QROWS * ROWF), 1)
    e = n % ROWF
    q = n // ROWF
    kk = 63 * q + e // 2
    selh = ((e % 2 == 0) & (m == kk)).astype(jnp.float32)
    selo = ((e % 2 == 1) & (m == kk)).astype(jnp.float32)
    z = jnp.zeros((NG, KPAD - NG), jnp.float32)
    for bs in range(_TBL_B1):
        hp = jnp.concatenate([h[bs], z], axis=-1)
        op = jnp.concatenate([o[bs], z], axis=-1)
        mm = (jnp.dot(hp, selh, preferred_element_type=jnp.float32)
              + jnp.dot(op, selo, preferred_element_type=jnp.float32))
        for qq in range(QROWS):
            out_ref[pl.ds((qq * NG + b * _TBL_B1 + bs) * NG, NG), :] = (
                mm[:, qq * ROWF:(qq + 1) * ROWF])


def _build_table(grid, opacity):
    return pl.pallas_call(
        _tbl_body,
        grid=(NG // _TBL_B1,),
        in_specs=[
            pl.BlockSpec((_TBL_B1, NG, NG * 9), lambda b: (b, 0, 0)),
            pl.BlockSpec((_TBL_B1, NG, NG), lambda b: (b, 0, 0)),
        ],
        out_specs=pl.BlockSpec((NIJ * QROWS, ROWF), lambda b: (0, 0)),
        out_shape=jax.ShapeDtypeStruct((NIJ * QROWS, ROWF), jnp.float32),
    )(grid.reshape(NG, NG, NG * 9), opacity)


def _sc_body(tbl, idx2, fpack, opack,
             idx_v, f_v, rows_v, o_v, sem_i, sem_f, sem_g):
    wid = lax.axis_index("s") * NC + lax.axis_index("c")
    npw = (opack.shape[0] // 2) // NW
    nch = npw // CHUNK

    def chunk_body(g, _):
        cg = wid * nch + g
        a_i = pltpu.async_copy(idx2.at[pl.ds(cg * CHUNK * 4, CHUNK * 4)],
                               idx_v, sem_i)
        a_f = pltpu.async_copy(fpack.at[pl.ds(cg * 768, 768)], f_v, sem_f)
        a_i.wait()
        copies = [
            pltpu.async_copy(tbl.at[idx_v.at[pl.ds(r * 128, 128)]],
                             rows_v.at[pl.ds(r * 128, 128)], sem_g)
            for r in range(GPER)
        ]
        a_f.wait()
        for c in copies:
            c.wait()

        def sub_body(t, _):
            sl = pl.ds(t * LANES, LANES)
            pv = f_v[pl.ds(5 * CHUNK + t * LANES, LANES)].astype(jnp.int32)
            fz = f_v[pl.ds(4 * CHUNK + t * LANES, LANES)]
            lanes = lax.broadcasted_iota(jnp.int32, (LANES,), 0)
            rbase = t * LANES + lanes
            zz = lanes * 0
            ev = pv + pv
            acc_h = jnp.zeros((LANES,), jnp.float32)
            acc_o = jnp.zeros((LANES,), jnp.float32)
            for c in range(4):
                rvec = c * CHUNK + rbase
                h0 = plsc.load_gather(rows_v, [rvec, zz, ev])
                h1 = plsc.load_gather(rows_v, [rvec, zz, ev + 2])
                o0 = plsc.load_gather(rows_v, [rvec, zz, ev + 1])
                o1 = plsc.load_gather(rows_v, [rvec, zz, ev + 3])
                w = f_v[pl.ds(c * CHUNK + t * LANES, LANES)]
                acc_h = acc_h + w * (h0 + fz * (h1 - h0))
                acc_o = acc_o + w * (o0 + fz * (o1 - o0))
            o_v[sl] = acc_h
            o_v[pl.ds(CHUNK + t * LANES, LANES)] = acc_o
            return _

        lax.fori_loop(0, CHUNK // LANES, sub_body, None)
        pltpu.sync_copy(o_v, opack.at[pl.ds(cg * 2 * CHUNK, 2 * CHUNK)])
        return _

    lax.fori_loop(0, nch, chunk_body, None)


def _sc_gather(tbl, idx2, fpack, n):
    mesh = plsc.VectorSubcoreMesh(core_axis_name="c", subcore_axis_name="s",
                                  num_cores=NC, num_subcores=NS)
    f = pl.kernel(
        _sc_body,
        out_type=jax.ShapeDtypeStruct((n * 2,), jnp.float32),
        mesh=mesh,
        scratch_types=[
            pltpu.VMEM((CHUNK * 4,), jnp.int32),
            pltpu.VMEM((CHUNK * 6,), jnp.float32),
            pltpu.VMEM((CHUNK * 4, 1, ROWF), jnp.float32),
            pltpu.VMEM((CHUNK * 2,), jnp.float32),
            pltpu.SemaphoreType.DMA,
            pltpu.SemaphoreType.DMA,
            pltpu.SemaphoreType.DMA,
        ],
        compiler_params=pltpu.CompilerParams(needs_layout_passes=False),
    )
    return f(tbl, idx2, fpack)


def _render_body(h_ref, o_ref, s_ref, out_ref):
    hh = h_ref[...]
    oo = o_ref[...]
    ss = s_ref[...]
    S1 = NB_SAMPLES - 1
    delta = ss[:, 1:] - ss[:, :S1]
    cur = delta * oo[:, :S1]
    t_i = lax.broadcasted_iota(jnp.int32, (S1, S1), 0)
    s_i = lax.broadcasted_iota(jnp.int32, (S1, S1), 1)
    excl = (t_i < s_i).astype(jnp.float32)
    cumm = jnp.dot(cur, excl, preferred_element_type=jnp.float32)
    trans = jnp.exp(-cumm)
    cs = 1.0 / (1.0 + jnp.exp(-hh[:, :S1]))
    res = jnp.sum(trans * (1.0 - jnp.exp(-cur)) * cs, axis=1)
    out_ref[...] = res.reshape(1, 1, -1)


def _render(interp_h, interp_o, samples, nb_rays):
    RB = 2048
    out = pl.pallas_call(
        _render_body,
        grid=(nb_rays // RB,),
        in_specs=[
            pl.BlockSpec((RB, NB_SAMPLES), lambda b: (b, 0)),
            pl.BlockSpec((RB, NB_SAMPLES), lambda b: (b, 0)),
            pl.BlockSpec((RB, NB_SAMPLES), lambda b: (b, 0)),
        ],
        out_specs=pl.BlockSpec((1, 1, RB), lambda b: (b, 0, 0)),
        out_shape=jax.ShapeDtypeStruct((nb_rays // RB, 1, RB), jnp.float32),
    )(interp_h, interp_o, samples)
    return out.reshape(nb_rays)


@jax.jit
def kernel(x, d, grid, opacity):
    nb_rays = x.shape[0]
    n = nb_rays * NB_SAMPLES

    inv_d = 1.0 / d
    inf = float(IDIM) * IDIM * IDIM
    t0 = (0.0 - x) * inv_d
    t1 = (float(IDIM) - x) * inv_d
    tmin = jnp.maximum(-inf, jnp.max(jnp.minimum(t0, t1), axis=1))
    tmax = jnp.minimum(inf, jnp.min(jnp.maximum(t0, t1), axis=1))
    u = jax.random.uniform(jax.random.key(42), (NB_SAMPLES, nb_rays),
                           jnp.float32)
    samples = jnp.sort((tmin[None, :] + u * (tmax - tmin)[None, :]).T, axis=1)

    px = x[:, 0:1] + samples * d[:, 0:1]
    py = x[:, 1:2] + samples * d[:, 1:2]
    pz = x[:, 2:3] + samples * d[:, 2:3]
    flx, fly, flz = jnp.floor(px), jnp.floor(py), jnp.floor(pz)
    bi = jnp.clip(flx.astype(jnp.int32), 0, IDIM - 1)
    bj = jnp.clip(fly.astype(jnp.int32), 0, IDIM - 1)
    bk = jnp.clip(flz.astype(jnp.int32), 0, IDIM - 1)
    fx, fy, fz = px - flx, py - fly, pz - flz
    q = bk // 63
    p = bk - 63 * q

    r00 = q * NIJ + bi * NG + bj
    idx2 = jnp.stack([r00.reshape(-1, CHUNK),
                      (r00 + 1).reshape(-1, CHUNK),
                      (r00 + NG).reshape(-1, CHUNK),
                      (r00 + NG + 1).reshape(-1, CHUNK)],
                     axis=1).reshape(n * 4)
    fpack = jnp.stack([((1 - fx) * (1 - fy)).reshape(-1, CHUNK),
                       ((1 - fx) * fy).reshape(-1, CHUNK),
                       (fx * (1 - fy)).reshape(-1, CHUNK),
                       (fx * fy).reshape(-1, CHUNK),
                       fz.reshape(-1, CHUNK),
                       p.astype(jnp.float32).reshape(-1, CHUNK)],
                      axis=1).reshape(n * 6)

    tbl = _build_table(grid, opacity).reshape(NIJ * QROWS, 1, ROWF)

    opack = _sc_gather(tbl, idx2, fpack, n).reshape(-1, 2, CHUNK)
    interp_h = opack[:, 0, :].reshape(nb_rays, NB_SAMPLES)
    interp_o = opack[:, 1, :].reshape(nb_rays, NB_SAMPLES)

    return _render(interp_h, interp_o, samples, nb_rays)
